# baseline (speedup 1.0000x reference)
import functools

import jax
import jax.numpy as jnp
from jax import lax
from jax.experimental import pallas as pl
from jax.experimental.pallas import tpu as pltpu
from jax.experimental.pallas import tpu_sc as plsc

B = 64
N = 1_000_000
K = 100
KPAD = 112
ACTW = 16
CT = 6
WC = 128 * CT
NFULL = (N // 128) // CT
NPAIRS = NFULL // 2
QPAIRS = 163
TAILC = N - NFULL * WC
CAP = 1024
CSTRIDE = CAP + 16
MCAP = 4096
FCAP = 128
NEG_INF = float("-inf")


def _lanes():
    return lax.iota(jnp.int32, 16)


def _splat_f32(x):
    return jnp.full((16,), x, dtype=jnp.float32)


def _splat_i32(x):
    return jnp.full((16,), x, dtype=jnp.int32)


def _mono_u32(x):
    b = plsc.bitcast(x, jnp.uint32)
    neg = b >= jnp.uint32(0x80000000)
    flip = jnp.where(neg, jnp.uint32(0xFFFFFFFF), jnp.uint32(0x80000000))
    return b ^ flip


def _sc_topk_sample(logits2d, tail_flat, gumbel_flat):
    mesh = plsc.VectorSubcoreMesh(core_axis_name="c", subcore_axis_name="s")

    @functools.partial(
        pl.kernel,
        out_type=(
            jax.ShapeDtypeStruct((B * KPAD,), jnp.float32),
            jax.ShapeDtypeStruct((B * ACTW,), jnp.int32),
        ),
        mesh=mesh,
        compiler_params=pltpu.CompilerParams(
            needs_layout_passes=False, use_tc_tiling_on_sc=True),
        scratch_types=[
            pltpu.VMEM((8, WC), jnp.float32),
            pltpu.VMEM((8, WC), jnp.float32),
            pltpu.VMEM((TAILC,), jnp.float32),
            pltpu.VMEM((8 * (CAP + 16),), jnp.float32),
            pltpu.VMEM((8 * (CAP + 16),), jnp.int32),
            pltpu.VMEM((8 * 16,), jnp.float32),
            pltpu.VMEM((8 * 16,), jnp.int32),
            pltpu.VMEM((16,), jnp.float32),
            pltpu.VMEM((MCAP,), jnp.float32),
            pltpu.VMEM((MCAP,), jnp.int32),
            pltpu.VMEM((FCAP,), jnp.float32),
            pltpu.VMEM((FCAP,), jnp.int32),
            pltpu.VMEM((KPAD,), jnp.float32),
            pltpu.VMEM((KPAD,), jnp.int32),
            pltpu.VMEM((KPAD,), jnp.float32),
            pltpu.VMEM((ACTW,), jnp.int32),
            pltpu.VMEM_SHARED((4 * 32 * 16,), jnp.float32),
            pltpu.VMEM_SHARED((4 * 32 * CAP,), jnp.float32),
            pltpu.VMEM_SHARED((4 * 32 * CAP,), jnp.int32),
            pltpu.SemaphoreType.DMA,
            pltpu.SemaphoreType.DMA,
        ],
    )
    def k(logits_hbm, tail_hbm, gumbel_hbm, vals_hbm, act_hbm,
          win0, win1, wtail, cv, ci, tvr, cntr, thst, mcv, mci, fv, fi,
          ov, oi, gb, actb, sh_th, sh_v, sh_i, sem0, sem1):
        lanes = _lanes()
        c = lax.axis_index("c")
        s = lax.axis_index("s")
        grp = s >> 2
        q = s & 3
        g8 = pl.multiple_of(c * 32 + grp * 8, 8)
        lrow0 = grp * 8

        def count_above(src_v, base, mid_u, cnt_v, nj):
            def cbody(j, acc):
                x = src_v[pl.ds(base + j * 16, 16)]
                valid = (j * 16 + lanes) < cnt_v
                m = jnp.logical_and(_mono_u32(x) > mid_u, valid)
                return acc + plsc.all_reduce_population_count(m)
            return lax.fori_loop(0, nj, cbody, _splat_i32(0))

        def find_threshold(src_v, base, cnt_v, hi_target):
            nj = (jnp.max(cnt_v) + 15) >> 4
            c0 = jnp.max(cnt_v)

            def cond(st):
                lo, hi, cc = st
                return jnp.logical_and(hi - lo > jnp.uint32(1),
                                       jnp.logical_or(cc < K,
                                                      cc > hi_target))

            def body(st):
                lo, hi, cc = st
                mid = lo + ((hi - lo) >> jnp.uint32(1))
                cm = jnp.max(count_above(src_v, base,
                                         jnp.full((16,), mid, jnp.uint32),
                                         cnt_v, nj))
                ok = cm >= K
                return (jnp.where(ok, mid, lo), jnp.where(ok, hi, mid),
                        jnp.where(ok, cm, cc))

            lo, _, _ = lax.while_loop(
                cond, body, (jnp.uint32(0), jnp.uint32(0xFFFFFFFF), c0))
            return jnp.full((16,), lo, jnp.uint32)

        def inv_mono(t_u):
            bv = t_u ^ jnp.where(t_u >= jnp.uint32(0x80000000),
                                 jnp.uint32(0x80000000),
                                 jnp.uint32(0xFFFFFFFF))
            return plsc.bitcast(bv, jnp.float32)

        def compact_row(sr):
            cnt_v = cntr[pl.ds(sr * 16, 16)]
            rb = sr * CSTRIDE
            t_u = find_threshold(cv, rb, cnt_v, 128)
            nj = (jnp.max(cnt_v) + 15) >> 4
            rbs = _splat_i32(1) * rb

            def cpbody(j, newcnt):
                x = cv[pl.ds(rb + j * 16, 16)]
                col = ci[pl.ds(rb + j * 16, 16)]
                valid = (j * 16 + lanes) < cnt_v
                m = jnp.logical_and(_mono_u32(x) > t_u, valid)
                ones = jnp.where(m, 1, 0).astype(jnp.int32)
                incl = plsc.cumsum(ones)
                pos = rbs + jnp.minimum(newcnt + incl - 1, CAP - 1)
                plsc.store_scatter(cv, [pos], x, mask=m)
                plsc.store_scatter(ci, [pos], col, mask=m)
                return jnp.minimum(
                    newcnt + plsc.all_reduce_population_count(m),
                    _splat_i32(CAP))

            newcnt = lax.fori_loop(0, nj, cpbody, _splat_i32(0))
            cntr[pl.ds(sr * 16, 16)] = newcnt
            tvr[pl.ds(sr * 16, 16)] = inv_mono(t_u)

        def append_run(load_fn, sr, col_v, nv):
            tvec = tvr[pl.ds(sr * 16, 16)]
            cnt_v = cntr[pl.ds(sr * 16, 16)]
            rbs = _splat_i32(1) * (sr * CSTRIDE)
            for i in range(nv):
                x = load_fn(i)
                m = x > tvec
                ones = jnp.where(m, 1, 0).astype(jnp.int32)
                incl = plsc.cumsum(ones)
                pos = rbs + jnp.minimum(cnt_v + incl - 1,
                                        _splat_i32(CAP + 15))
                plsc.store_scatter(cv, [pos], x, mask=m)
                plsc.store_scatter(ci, [pos], col_v + i * 16, mask=m)
                cnt_v = jnp.minimum(
                    cnt_v + plsc.all_reduce_population_count(m),
                    _splat_i32(CAP))
            cntr[pl.ds(sr * 16, 16)] = cnt_v

            @pl.when(cnt_v[0] >= CAP - 16 * nv)
            def _():
                compact_row(sr)

        def run_max(load_fn, nv):
            vs = [load_fn(i) for i in range(nv)]
            while len(vs) > 1:
                nxt = [jnp.maximum(vs[i], vs[i + 1])
                       for i in range(0, len(vs) - 1, 2)]
                if len(vs) % 2:
                    nxt.append(vs[-1])
                vs = nxt
            return vs[0]

        def scan_pair(win, sr0, cw, col_v):
            l0 = lambda i: win[sr0, pl.ds(cw + i * 16, 16)]
            l1 = lambda i: win[sr0 + 1, pl.ds(cw + i * 16, 16)]
            hit = jnp.logical_or(run_max(l0, 8) > tvr[pl.ds(sr0 * 16, 16)],
                                 run_max(l1, 8) > tvr[pl.ds((sr0 + 1) * 16, 16)])

            @pl.when(jnp.any(hit))
            def _():
                @pl.when(jnp.any(run_max(l0, 8) > tvr[pl.ds(sr0 * 16, 16)]))
                def _():
                    append_run(l0, sr0, col_v, 8)

                @pl.when(jnp.any(run_max(l1, 8) > tvr[pl.ds((sr0 + 1) * 16, 16)]))
                def _():
                    append_run(l1, sr0 + 1, col_v, 8)

        def scan_window(win, c0):
            def tbody(tp, _):
                t = tp >> 2
                sr0 = (tp & 3) * 2
                cw = t * 128
                scan_pair(win, sr0, cw, c0 + cw + lanes)
                return 0
            lax.fori_loop(0, CT * 4, tbody, 0)

        def src_at(w):
            col0 = pl.multiple_of(w * WC, 128)
            return logits_hbm.at[pl.ds(g8, 8), pl.ds(col0, WC)]

        def init_row(sr, _):
            tvr[pl.ds(sr * 16, 16)] = _splat_f32(NEG_INF)
            cntr[pl.ds(sr * 16, 16)] = _splat_i32(0)
            return 0
        lax.fori_loop(0, 8, init_row, 0)

        wq0 = q * (2 * QPAIRS)
        pend = jnp.minimum((q + 1) * QPAIRS, NPAIRS)

        pltpu.sync_copy(src_at(wq0), win0)
        scan_window(win0, wq0 * WC)

        def seed_row(sr, _):
            compact_row(sr)
            thst[...] = tvr[pl.ds(sr * 16, 16)]
            pltpu.sync_copy(thst, sh_th.at[pl.ds((q * 32 + lrow0 + sr) * 16, 16)])
            cntr[pl.ds(sr * 16, 16)] = _splat_i32(0)
            return 0
        lax.fori_loop(0, 8, seed_row, 0)
        plsc.subcore_barrier()

        def merge_th(sr, _):
            tv = tvr[pl.ds(sr * 16, 16)]
            for q2 in range(4):
                pltpu.sync_copy(sh_th.at[pl.ds((q2 * 32 + lrow0 + sr) * 16, 16)], thst)
                tv = jnp.maximum(tv, thst[...])
            tvr[pl.ds(sr * 16, 16)] = tv
            return 0
        lax.fori_loop(0, 8, merge_th, 0)

        pltpu.make_async_copy(src_at(wq0), win0, sem0).start()

        def wbody(p, _):
            w0 = 2 * p
            pltpu.make_async_copy(src_at(w0 + 1), win1, sem1).start()
            pltpu.make_async_copy(src_at(w0), win0, sem0).wait()
            scan_window(win0, w0 * WC)

            @pl.when(p < pend - 1)
            def _():
                pltpu.make_async_copy(src_at(w0 + 2), win0, sem0).start()

            pltpu.make_async_copy(src_at(w0 + 1), win1, sem1).wait()
            scan_window(win1, (w0 + 1) * WC)
            return 0

        lax.fori_loop(q * QPAIRS, pend, wbody, 0)

        @pl.when(q == 3)
        def _():
            def trow(sr, _):
                pltpu.sync_copy(
                    tail_hbm.at[pl.ds((g8 + sr) * TAILC, TAILC)], wtail)
                tl = lambda i: wtail[pl.ds(i * 16, 16)]
                tcol = NFULL * WC + lanes

                @pl.when(jnp.any(run_max(tl, TAILC // 16) > tvr[pl.ds(sr * 16, 16)]))
                def _():
                    append_run(tl, sr, tcol, TAILC // 16)
                return 0
            lax.fori_loop(0, 8, trow, 0)

        def pub_row(sr, _):
            cnt_v = cntr[pl.ds(sr * 16, 16)]
            rbs = _splat_i32(1) * (sr * CSTRIDE)

            def padj(j, _):
                posp = j * 16 + lanes
                plsc.store_scatter(cv, [rbs + posp], _splat_f32(NEG_INF),
                                   mask=posp >= cnt_v)
                return 0
            lax.fori_loop(0, CAP // 16, padj, 0)
            shb = (q * 32 + lrow0 + sr) * CAP
            pltpu.sync_copy(cv.at[pl.ds(sr * CSTRIDE, CAP)],
                            sh_v.at[pl.ds(shb, CAP)])
            pltpu.sync_copy(ci.at[pl.ds(sr * CSTRIDE, CAP)],
                            sh_i.at[pl.ds(shb, CAP)])
            return 0
        lax.fori_loop(0, 8, pub_row, 0)
        plsc.subcore_barrier()

        for lr in range(2):
            lrow = s * 2 + lr
            row = c * 32 + lrow
            for q2 in range(4):
                pltpu.sync_copy(sh_v.at[pl.ds((q2 * 32 + lrow) * CAP, CAP)],
                                mcv.at[pl.ds(q2 * CAP, CAP)])
                pltpu.sync_copy(sh_i.at[pl.ds((q2 * 32 + lrow) * CAP, CAP)],
                                mci.at[pl.ds(q2 * CAP, CAP)])
            mcnt = _splat_i32(MCAP)
            t_u = find_threshold(mcv, 0, mcnt, FCAP - 8)
            for i in range(FCAP // 16):
                fv[pl.ds(i * 16, 16)] = _splat_f32(NEG_INF)
                fi[pl.ds(i * 16, 16)] = _splat_i32(0)

            def fcbody(j, newcnt):
                x = mcv[pl.ds(j * 16, 16)]
                col = mci[pl.ds(j * 16, 16)]
                m = _mono_u32(x) > t_u
                ones = jnp.where(m, 1, 0).astype(jnp.int32)
                incl = plsc.cumsum(ones)
                pos = jnp.minimum(newcnt + incl - 1, FCAP - 1)
                plsc.store_scatter(fv, [pos], x, mask=m)
                plsc.store_scatter(fi, [pos], col, mask=m)
                return jnp.minimum(
                    newcnt + plsc.all_reduce_population_count(m),
                    _splat_i32(FCAP))
            lax.fori_loop(0, MCAP // 16, fcbody, _splat_i32(0))

            for i in range(KPAD // 16):
                ov[pl.ds(i * 16, 16)] = _splat_f32(NEG_INF)
                oi[pl.ds(i * 16, 16)] = _splat_i32(0)

            def ebody(j, _):
                mvec = fv[pl.ds(0, 16)]
                for i in range(1, FCAP // 16):
                    mvec = jnp.maximum(mvec, fv[pl.ds(i * 16, 16)])
                msp = _splat_f32(1.0) * jnp.max(mvec)
                pos_v = _splat_i32(FCAP)
                for i in range(FCAP // 16):
                    eq = fv[pl.ds(i * 16, 16)] == msp
                    pos_v = jnp.minimum(
                        pos_v, jnp.where(eq, i * 16 + lanes, FCAP))
                pos = _splat_i32(1) * jnp.min(pos_v)
                pos = jnp.minimum(pos, _splat_i32(FCAP - 1))
                idv = plsc.load_gather(fi, [pos])
                jsp = _splat_i32(1) * j
                lane0 = lanes == 0
                plsc.store_scatter(ov, [jsp], msp, mask=lane0)
                plsc.store_scatter(oi, [jsp], idv, mask=lane0)
                plsc.store_scatter(fv, [pos], _splat_f32(NEG_INF),
                                   mask=lane0)
                return 0
            lax.fori_loop(0, K, ebody, 0)

            pltpu.sync_copy(gumbel_hbm.at[pl.ds(row * KPAD, KPAD)], gb)
            zbest = _splat_f32(NEG_INF)
            zs = []
            for i in range(KPAD // 16):
                z = ov[pl.ds(i * 16, 16)] + gb[pl.ds(i * 16, 16)]
                zs.append(z)
                zbest = jnp.maximum(zbest, z)
            msp = _splat_f32(1.0) * jnp.max(zbest)
            pos_v = _splat_i32(KPAD)
            for i in range(KPAD // 16):
                eq = zs[i] == msp
                pos_v = jnp.minimum(pos_v,
                                    jnp.where(eq, i * 16 + lanes, KPAD))
            pos = _splat_i32(1) * jnp.min(pos_v)
            pos = jnp.minimum(pos, _splat_i32(KPAD - 1))
            actb[...] = plsc.load_gather(oi, [pos])

            pltpu.sync_copy(ov, vals_hbm.at[pl.ds(row * KPAD, KPAD)])
            pltpu.sync_copy(actb, act_hbm.at[pl.ds(row * ACTW, ACTW)])

    return k(logits2d, tail_flat, gumbel_flat)


def kernel(logits):
    gkey = jax.random.key(42)
    u = jax.random.uniform(gkey, (B, K), minval=1e-20, maxval=1.0)
    gumbel = -jnp.log(-jnp.log(u))
    gpad = jnp.full((B, KPAD), NEG_INF, dtype=jnp.float32)
    gpad = gpad.at[:, :K].set(gumbel)

    tail_flat = logits[:, NFULL * WC:].reshape(-1)
    vals_flat, act_flat = _sc_topk_sample(logits, tail_flat,
                                          gpad.reshape(-1))
    vals = vals_flat.reshape(B, KPAD)[:, :K]
    act = act_flat.reshape(B, ACTW)[:, 0]
    return act, vals


# speedup vs baseline: 1.3074x; 1.3074x over previous
import functools

import jax
import jax.numpy as jnp
from jax import lax
from jax.experimental import pallas as pl
from jax.experimental.pallas import tpu as pltpu
from jax.experimental.pallas import tpu_sc as plsc

B = 64
N = 1_000_000
K = 100
KPAD = 112
ACTW = 16
CT = 6
WC = 128 * CT
NFULL = (N // 128) // CT
NWIN_PAIRS = NFULL // 2
TAILC = N - NFULL * WC
CAP = 4096
FCAP = 128
NEG_INF = float("-inf")


def _lanes():
    return lax.iota(jnp.int32, 16)


def _splat_f32(x):
    return jnp.full((16,), x, dtype=jnp.float32)


def _splat_i32(x):
    return jnp.full((16,), x, dtype=jnp.int32)


def _mono_u32(x):
    b = plsc.bitcast(x, jnp.uint32)
    neg = b >= jnp.uint32(0x80000000)
    flip = jnp.where(neg, jnp.uint32(0xFFFFFFFF), jnp.uint32(0x80000000))
    return b ^ flip


def _sc_topk_sample(logits2d, tail_flat, gumbel_flat):
    mesh = plsc.VectorSubcoreMesh(core_axis_name="c", subcore_axis_name="s")

    @functools.partial(
        pl.kernel,
        out_type=(
            jax.ShapeDtypeStruct((B * KPAD,), jnp.float32),
            jax.ShapeDtypeStruct((B * ACTW,), jnp.int32),
        ),
        mesh=mesh,
        compiler_params=pltpu.CompilerParams(
            needs_layout_passes=False, use_tc_tiling_on_sc=True),
        scratch_types=[
            pltpu.VMEM((8, WC), jnp.float32),
            pltpu.VMEM((8, WC), jnp.float32),
            pltpu.VMEM((TAILC,), jnp.float32),
            pltpu.VMEM((CAP + 16,), jnp.float32),
            pltpu.VMEM((CAP + 16,), jnp.int32),
            pltpu.VMEM((CAP + 16,), jnp.float32),
            pltpu.VMEM((CAP + 16,), jnp.int32),
            pltpu.VMEM((FCAP,), jnp.float32),
            pltpu.VMEM((FCAP,), jnp.int32),
            pltpu.VMEM((KPAD,), jnp.float32),
            pltpu.VMEM((KPAD,), jnp.int32),
            pltpu.VMEM((KPAD,), jnp.float32),
            pltpu.VMEM((ACTW,), jnp.int32),
            pltpu.SemaphoreType.DMA,
            pltpu.SemaphoreType.DMA,
        ],
    )
    def k(logits_hbm, tail_hbm, gumbel_hbm, vals_hbm, act_hbm,
          win0, win1, wtail, cv0, ci0, cv1, ci1, fv, fi, ov, oi, gb, actb,
          sem0, sem1):
        lanes = _lanes()
        wid = lax.axis_index("s") * 2 + lax.axis_index("c")
        row0 = wid * 2
        g8 = pl.multiple_of((wid >> 2) * 8, 8)
        sub0 = (row0 % 8)
        cvs = (cv0, cv1)
        cis = (ci0, ci1)

        def count_above(src_v, mid_u, cnt_v, nj):
            def cbody(j, acc):
                x = src_v[pl.ds(j * 16, 16)]
                u = _mono_u32(x)
                valid = (j * 16 + lanes) < cnt_v
                m = jnp.logical_and(u > mid_u, valid)
                return acc + plsc.all_reduce_population_count(m)
            return lax.fori_loop(0, nj, cbody, _splat_i32(0))

        def find_threshold(src_v, cnt_v, hi_target):
            nj = (jnp.max(cnt_v) + 15) >> 4
            c0 = jnp.max(cnt_v)

            def cond(s):
                lo, hi, c = s
                return jnp.logical_and(hi - lo > jnp.uint32(1),
                                       jnp.logical_or(c < K, c > hi_target))

            def body(s):
                lo, hi, c = s
                mid = lo + ((hi - lo) >> jnp.uint32(1))
                cm = jnp.max(count_above(src_v,
                                         jnp.full((16,), mid, jnp.uint32),
                                         cnt_v, nj))
                ok = cm >= K
                return (jnp.where(ok, mid, lo), jnp.where(ok, hi, mid),
                        jnp.where(ok, cm, c))

            lo, _, _ = lax.while_loop(
                cond, body, (jnp.uint32(0), jnp.uint32(0xFFFFFFFF), c0))
            return jnp.full((16,), lo, jnp.uint32)

        def compress_into(src_v, src_i, dst_v, dst_i, dcap, t_u, cnt_v):
            nj = (jnp.max(cnt_v) + 15) >> 4

            def cpbody(j, newcnt):
                x = src_v[pl.ds(j * 16, 16)]
                col = src_i[pl.ds(j * 16, 16)]
                valid = (j * 16 + lanes) < cnt_v
                m = jnp.logical_and(_mono_u32(x) > t_u, valid)
                ones = jnp.where(m, 1, 0).astype(jnp.int32)
                incl = plsc.cumsum(ones)
                pos = jnp.minimum(newcnt + incl - 1, dcap - 1)
                plsc.store_scatter(dst_v, [pos], x, mask=m)
                plsc.store_scatter(dst_i, [pos], col, mask=m)
                return jnp.minimum(
                    newcnt + plsc.all_reduce_population_count(m),
                    _splat_i32(dcap))
            return lax.fori_loop(0, nj, cpbody, _splat_i32(0))

        def make_compact(r):
            def compact(cnt_v, tvec):
                t_u = find_threshold(cvs[r], cnt_v, 128)
                newcnt = compress_into(cvs[r], cis[r], cvs[r], cis[r],
                                       CAP, t_u, cnt_v)
                bv = t_u ^ jnp.where(t_u >= jnp.uint32(0x80000000),
                                     jnp.uint32(0x80000000),
                                     jnp.uint32(0xFFFFFFFF))
                return newcnt, plsc.bitcast(bv, jnp.float32)
            return compact

        compact_fns = (make_compact(0), make_compact(1))

        def append_chunk(win, r, sub, cw, col_v, nv, st):
            cnt_v, tvec = st
            for i in range(nv):
                x = win[sub, pl.ds(cw + i * 16, 16)]
                m = x > tvec
                ones = jnp.where(m, 1, 0).astype(jnp.int32)
                incl = plsc.cumsum(ones)
                pos = jnp.minimum(cnt_v + incl - 1, _splat_i32(CAP + 15))
                plsc.store_scatter(cvs[r], [pos], x, mask=m)
                plsc.store_scatter(cis[r], [pos], col_v + i * 16, mask=m)
                cnt_v = jnp.minimum(
                    cnt_v + plsc.all_reduce_population_count(m),
                    _splat_i32(CAP))
            return lax.cond(cnt_v[0] >= CAP - 16 * nv, compact_fns[r],
                            lambda c, t: (c, t), cnt_v, tvec)

        def chunk_max(win, sub, cw, nv):
            mv = win[sub, pl.ds(cw, 16)]
            for i in range(1, nv):
                mv = jnp.maximum(mv, win[sub, pl.ds(cw + i * 16, 16)])
            return mv

        def scan_chunk(win, r, sub, cw, col_v, nv, st):
            return lax.cond(
                jnp.any(chunk_max(win, sub, cw, nv) > st[1]),
                lambda st: append_chunk(win, r, sub, cw, col_v, nv, st),
                lambda st: st, st)

        def scan_tail(buf, r, col_v, st):
            cnt_v, tvec = st
            anym = buf[pl.ds(0, 16)] > tvec
            for i in range(1, TAILC // 16):
                anym = jnp.logical_or(anym, buf[pl.ds(i * 16, 16)] > tvec)

            def slow(cnt_v):
                for i in range(TAILC // 16):
                    x = buf[pl.ds(i * 16, 16)]
                    m = x > tvec
                    ones = jnp.where(m, 1, 0).astype(jnp.int32)
                    incl = plsc.cumsum(ones)
                    pos = jnp.minimum(cnt_v + incl - 1, _splat_i32(CAP + 15))
                    plsc.store_scatter(cvs[r], [pos], x, mask=m)
                    plsc.store_scatter(cis[r], [pos], col_v + i * 16, mask=m)
                    cnt_v = jnp.minimum(
                        cnt_v + plsc.all_reduce_population_count(m),
                        _splat_i32(CAP))
                return cnt_v

            cnt_v = lax.cond(jnp.any(anym), slow, lambda c: c, cnt_v)
            cnt_v, tvec = lax.cond(
                cnt_v[0] >= CAP - TAILC, compact_fns[r],
                lambda c, t: (c, t), cnt_v, tvec)
            return cnt_v, tvec

        def scan_window(win, c0, st0, st1):
            def tbody(t, carry):
                st0, st1 = carry
                cw = t * 128
                hit = jnp.logical_or(
                    chunk_max(win, sub0, cw, 8) > st0[1],
                    chunk_max(win, sub0 + 1, cw, 8) > st1[1])

                def slow(carry):
                    st0, st1 = carry
                    col_v = c0 + cw + lanes
                    st0 = scan_chunk(win, 0, sub0, cw, col_v, 8, st0)
                    st1 = scan_chunk(win, 1, sub0 + 1, cw, col_v, 8, st1)
                    return st0, st1

                return lax.cond(jnp.any(hit), slow, lambda c: c,
                                (st0, st1))
            return lax.fori_loop(0, CT, tbody, (st0, st1))

        def src_at(w):
            c0 = pl.multiple_of(w * WC, 128)
            return logits_hbm.at[pl.ds(g8, 8), pl.ds(c0, WC)]

        pltpu.make_async_copy(src_at(0), win0, sem0).start()

        def wbody(i, carry):
            st0, st1 = carry
            w0 = 2 * i
            pltpu.make_async_copy(src_at(w0 + 1), win1, sem1).start()
            pltpu.make_async_copy(src_at(w0), win0, sem0).wait()
            st0, st1 = scan_window(win0, w0 * WC, st0, st1)

            @pl.when(i < NWIN_PAIRS - 1)
            def _():
                pltpu.make_async_copy(src_at(w0 + 2), win0, sem0).start()

            pltpu.make_async_copy(src_at(w0 + 1), win1, sem1).wait()
            st0, st1 = scan_window(win1, (w0 + 1) * WC, st0, st1)
            return st0, st1

        init = ((_splat_i32(0), _splat_f32(NEG_INF)),
                (_splat_i32(0), _splat_f32(NEG_INF)))
        (st0, st1) = lax.fori_loop(0, NWIN_PAIRS, wbody, init)

        tcol = NFULL * WC + lanes
        pltpu.sync_copy(tail_hbm.at[pl.ds(row0 * TAILC, TAILC)], wtail)
        st0 = scan_tail(wtail, 0, tcol, st0)
        pltpu.sync_copy(tail_hbm.at[pl.ds((row0 + 1) * TAILC, TAILC)], wtail)
        st1 = scan_tail(wtail, 1, tcol, st1)

        for r in range(2):
            row = row0 + r
            cnt_v = (st0, st1)[r][0]
            t_u = find_threshold(cvs[r], cnt_v, FCAP - 8)
            for i in range(FCAP // 16):
                fv[pl.ds(i * 16, 16)] = _splat_f32(NEG_INF)
                fi[pl.ds(i * 16, 16)] = _splat_i32(0)
            compress_into(cvs[r], cis[r], fv, fi, FCAP, t_u, cnt_v)
            for i in range(KPAD // 16):
                ov[pl.ds(i * 16, 16)] = _splat_f32(NEG_INF)
                oi[pl.ds(i * 16, 16)] = _splat_i32(0)

            def ebody(j, _):
                mvec = fv[pl.ds(0, 16)]
                for i in range(1, FCAP // 16):
                    mvec = jnp.maximum(mvec, fv[pl.ds(i * 16, 16)])
                msp = _splat_f32(1.0) * jnp.max(mvec)
                pos_v = _splat_i32(FCAP)
                for i in range(FCAP // 16):
                    eq = fv[pl.ds(i * 16, 16)] == msp
                    pos_v = jnp.minimum(
                        pos_v, jnp.where(eq, i * 16 + lanes, FCAP))
                pos = _splat_i32(1) * jnp.min(pos_v)
                pos = jnp.minimum(pos, _splat_i32(FCAP - 1))
                idv = plsc.load_gather(fi, [pos])
                jsp = _splat_i32(1) * j
                lane0 = lanes == 0
                plsc.store_scatter(ov, [jsp], msp, mask=lane0)
                plsc.store_scatter(oi, [jsp], idv, mask=lane0)
                plsc.store_scatter(fv, [pos], _splat_f32(NEG_INF),
                                   mask=lane0)
                return 0
            lax.fori_loop(0, K, ebody, 0)

            pltpu.sync_copy(gumbel_hbm.at[pl.ds(row * KPAD, KPAD)], gb)
            zbest = _splat_f32(NEG_INF)
            zs = []
            for i in range(KPAD // 16):
                z = ov[pl.ds(i * 16, 16)] + gb[pl.ds(i * 16, 16)]
                zs.append(z)
                zbest = jnp.maximum(zbest, z)
            msp = _splat_f32(1.0) * jnp.max(zbest)
            pos_v = _splat_i32(KPAD)
            for i in range(KPAD // 16):
                eq = zs[i] == msp
                pos_v = jnp.minimum(pos_v,
                                    jnp.where(eq, i * 16 + lanes, KPAD))
            pos = _splat_i32(1) * jnp.min(pos_v)
            pos = jnp.minimum(pos, _splat_i32(KPAD - 1))
            actb[...] = plsc.load_gather(oi, [pos])

            pltpu.sync_copy(ov, vals_hbm.at[pl.ds(row * KPAD, KPAD)])
            pltpu.sync_copy(actb, act_hbm.at[pl.ds(row * ACTW, ACTW)])

    return k(logits2d, tail_flat, gumbel_flat)


def kernel(logits):
    gkey = jax.random.key(42)
    u = jax.random.uniform(gkey, (B, K), minval=1e-20, maxval=1.0)
    gumbel = -jnp.log(-jnp.log(u))
    gpad = jnp.full((B, KPAD), NEG_INF, dtype=jnp.float32)
    gpad = gpad.at[:, :K].set(gumbel)

    tail_flat = logits[:, NFULL * WC:].reshape(-1)
    vals_flat, act_flat = _sc_topk_sample(logits, tail_flat,
                                          gpad.reshape(-1))
    vals = vals_flat.reshape(B, KPAD)[:, :K]
    act = act_flat.reshape(B, ACTW)[:, 0]
    return act, vals


# speedup vs baseline: 1.3330x; 1.0196x over previous
import functools

import jax
import jax.numpy as jnp
from jax import lax
from jax.experimental import pallas as pl
from jax.experimental.pallas import tpu as pltpu
from jax.experimental.pallas import tpu_sc as plsc

B = 64
N = 1_000_000
K = 100
KPAD = 112
ACTW = 16
CT = 6
WC = 128 * CT
NFULL = (N // 128) // CT
NWIN_PAIRS = NFULL // 2
TAILC = N - NFULL * WC
CAP = 4096
FCAP = 128
NEG_INF = float("-inf")


def _lanes():
    return lax.iota(jnp.int32, 16)


def _splat_f32(x):
    return jnp.full((16,), x, dtype=jnp.float32)


def _splat_i32(x):
    return jnp.full((16,), x, dtype=jnp.int32)


def _mono_u32(x):
    b = plsc.bitcast(x, jnp.uint32)
    neg = b >= jnp.uint32(0x80000000)
    flip = jnp.where(neg, jnp.uint32(0xFFFFFFFF), jnp.uint32(0x80000000))
    return b ^ flip


def _sc_topk_sample(logits2d, tail_flat, gumbel_flat):
    mesh = plsc.VectorSubcoreMesh(core_axis_name="c", subcore_axis_name="s")

    @functools.partial(
        pl.kernel,
        out_type=(
            jax.ShapeDtypeStruct((B * KPAD,), jnp.float32),
            jax.ShapeDtypeStruct((B * ACTW,), jnp.int32),
        ),
        mesh=mesh,
        compiler_params=pltpu.CompilerParams(
            needs_layout_passes=False, use_tc_tiling_on_sc=True),
        scratch_types=[
            pltpu.VMEM((8, WC), jnp.float32),
            pltpu.VMEM((8, WC), jnp.float32),
            pltpu.VMEM((TAILC,), jnp.float32),
            pltpu.VMEM((CAP + 16,), jnp.float32),
            pltpu.VMEM((CAP + 16,), jnp.int32),
            pltpu.VMEM((CAP + 16,), jnp.float32),
            pltpu.VMEM((CAP + 16,), jnp.int32),
            pltpu.VMEM((FCAP,), jnp.float32),
            pltpu.VMEM((FCAP,), jnp.int32),
            pltpu.VMEM((KPAD,), jnp.float32),
            pltpu.VMEM((KPAD,), jnp.int32),
            pltpu.VMEM((KPAD,), jnp.float32),
            pltpu.VMEM((ACTW,), jnp.int32),
            pltpu.SemaphoreType.DMA,
            pltpu.SemaphoreType.DMA,
        ],
    )
    def k(logits_hbm, tail_hbm, gumbel_hbm, vals_hbm, act_hbm,
          win0, win1, wtail, cv0, ci0, cv1, ci1, fv, fi, ov, oi, gb, actb,
          sem0, sem1):
        lanes = _lanes()
        wid = lax.axis_index("s") * 2 + lax.axis_index("c")
        row0 = wid * 2
        g8 = pl.multiple_of((wid >> 2) * 8, 8)
        sub0 = (row0 % 8)
        cvs = (cv0, cv1)
        cis = (ci0, ci1)

        def count_above(src_v, mid_u, cnt_v, nj):
            def cbody(j, acc):
                x = src_v[pl.ds(j * 16, 16)]
                u = _mono_u32(x)
                valid = (j * 16 + lanes) < cnt_v
                m = jnp.logical_and(u > mid_u, valid)
                return acc + plsc.all_reduce_population_count(m)
            return lax.fori_loop(0, nj, cbody, _splat_i32(0))

        def find_threshold(src_v, cnt_v, hi_target):
            nj = (jnp.max(cnt_v) + 15) >> 4
            c0 = jnp.max(cnt_v)

            def cond(s):
                lo, hi, c = s
                return jnp.logical_and(hi - lo > jnp.uint32(1),
                                       jnp.logical_or(c < K, c > hi_target))

            def body(s):
                lo, hi, c = s
                mid = lo + ((hi - lo) >> jnp.uint32(1))
                cm = jnp.max(count_above(src_v,
                                         jnp.full((16,), mid, jnp.uint32),
                                         cnt_v, nj))
                ok = cm >= K
                return (jnp.where(ok, mid, lo), jnp.where(ok, hi, mid),
                        jnp.where(ok, cm, c))

            lo, _, _ = lax.while_loop(
                cond, body, (jnp.uint32(0), jnp.uint32(0xFFFFFFFF), c0))
            return jnp.full((16,), lo, jnp.uint32)

        def compress_into(src_v, src_i, dst_v, dst_i, dcap, t_u, cnt_v):
            nj = (jnp.max(cnt_v) + 15) >> 4

            def cpbody(j, newcnt):
                x = src_v[pl.ds(j * 16, 16)]
                col = src_i[pl.ds(j * 16, 16)]
                valid = (j * 16 + lanes) < cnt_v
                m = jnp.logical_and(_mono_u32(x) > t_u, valid)
                ones = jnp.where(m, 1, 0).astype(jnp.int32)
                incl = plsc.cumsum(ones)
                pos = jnp.minimum(newcnt + incl - 1, dcap - 1)
                plsc.store_scatter(dst_v, [pos], x, mask=m)
                plsc.store_scatter(dst_i, [pos], col, mask=m)
                return jnp.minimum(
                    newcnt + plsc.all_reduce_population_count(m),
                    _splat_i32(dcap))
            return lax.fori_loop(0, nj, cpbody, _splat_i32(0))

        def make_compact(r):
            def compact(cnt_v, tvec):
                t_u = find_threshold(cvs[r], cnt_v, 128)
                newcnt = compress_into(cvs[r], cis[r], cvs[r], cis[r],
                                       CAP, t_u, cnt_v)
                bv = t_u ^ jnp.where(t_u >= jnp.uint32(0x80000000),
                                     jnp.uint32(0x80000000),
                                     jnp.uint32(0xFFFFFFFF))
                return newcnt, plsc.bitcast(bv, jnp.float32)
            return compact

        compact_fns = (make_compact(0), make_compact(1))

        def scan_chunk(win, r, sub, cw, col_v, nv, st):
            cnt_v, tvec = st
            anym = win[sub, pl.ds(cw, 16)] > tvec
            for i in range(1, nv):
                anym = jnp.logical_or(
                    anym, win[sub, pl.ds(cw + i * 16, 16)] > tvec)

            def slow(cnt_v):
                for i in range(nv):
                    x = win[sub, pl.ds(cw + i * 16, 16)]
                    m = x > tvec
                    ones = jnp.where(m, 1, 0).astype(jnp.int32)
                    incl = plsc.cumsum(ones)
                    pos = jnp.minimum(cnt_v + incl - 1, _splat_i32(CAP + 15))
                    plsc.store_scatter(cvs[r], [pos], x, mask=m)
                    plsc.store_scatter(cis[r], [pos], col_v + i * 16, mask=m)
                    cnt_v = jnp.minimum(
                        cnt_v + plsc.all_reduce_population_count(m),
                        _splat_i32(CAP))
                return cnt_v

            cnt_v = lax.cond(jnp.any(anym), slow, lambda c: c, cnt_v)
            cnt_v, tvec = lax.cond(
                cnt_v[0] >= CAP - 16 * nv, compact_fns[r],
                lambda c, t: (c, t), cnt_v, tvec)
            return cnt_v, tvec

        def scan_tail(buf, r, col_v, st):
            cnt_v, tvec = st
            anym = buf[pl.ds(0, 16)] > tvec
            for i in range(1, TAILC // 16):
                anym = jnp.logical_or(anym, buf[pl.ds(i * 16, 16)] > tvec)

            def slow(cnt_v):
                for i in range(TAILC // 16):
                    x = buf[pl.ds(i * 16, 16)]
                    m = x > tvec
                    ones = jnp.where(m, 1, 0).astype(jnp.int32)
                    incl = plsc.cumsum(ones)
                    pos = jnp.minimum(cnt_v + incl - 1, _splat_i32(CAP + 15))
                    plsc.store_scatter(cvs[r], [pos], x, mask=m)
                    plsc.store_scatter(cis[r], [pos], col_v + i * 16, mask=m)
                    cnt_v = jnp.minimum(
                        cnt_v + plsc.all_reduce_population_count(m),
                        _splat_i32(CAP))
                return cnt_v

            cnt_v = lax.cond(jnp.any(anym), slow, lambda c: c, cnt_v)
            cnt_v, tvec = lax.cond(
                cnt_v[0] >= CAP - TAILC, compact_fns[r],
                lambda c, t: (c, t), cnt_v, tvec)
            return cnt_v, tvec

        def scan_window(win, c0, st0, st1):
            def tbody(t, carry):
                st0, st1 = carry
                col_v = c0 + t * 128 + lanes
                st0 = scan_chunk(win, 0, sub0, t * 128, col_v, 8, st0)
                st1 = scan_chunk(win, 1, sub0 + 1, t * 128, col_v, 8, st1)
                return st0, st1
            return lax.fori_loop(0, CT, tbody, (st0, st1))

        def src_at(w):
            c0 = pl.multiple_of(w * WC, 128)
            return logits_hbm.at[pl.ds(g8, 8), pl.ds(c0, WC)]

        pltpu.make_async_copy(src_at(0), win0, sem0).start()

        def wbody(i, carry):
            st0, st1 = carry
            w0 = 2 * i
            pltpu.make_async_copy(src_at(w0 + 1), win1, sem1).start()
            pltpu.make_async_copy(src_at(w0), win0, sem0).wait()
            st0, st1 = scan_window(win0, w0 * WC, st0, st1)

            @pl.when(i < NWIN_PAIRS - 1)
            def _():
                pltpu.make_async_copy(src_at(w0 + 2), win0, sem0).start()

            pltpu.make_async_copy(src_at(w0 + 1), win1, sem1).wait()
            st0, st1 = scan_window(win1, (w0 + 1) * WC, st0, st1)
            return st0, st1

        init = ((_splat_i32(0), _splat_f32(NEG_INF)),
                (_splat_i32(0), _splat_f32(NEG_INF)))
        (st0, st1) = lax.fori_loop(0, NWIN_PAIRS, wbody, init)

        tcol = NFULL * WC + lanes
        pltpu.sync_copy(tail_hbm.at[pl.ds(row0 * TAILC, TAILC)], wtail)
        st0 = scan_tail(wtail, 0, tcol, st0)
        pltpu.sync_copy(tail_hbm.at[pl.ds((row0 + 1) * TAILC, TAILC)], wtail)
        st1 = scan_tail(wtail, 1, tcol, st1)

        for r in range(2):
            row = row0 + r
            cnt_v = (st0, st1)[r][0]
            t_u = find_threshold(cvs[r], cnt_v, FCAP - 8)
            for i in range(FCAP // 16):
                fv[pl.ds(i * 16, 16)] = _splat_f32(NEG_INF)
                fi[pl.ds(i * 16, 16)] = _splat_i32(0)
            compress_into(cvs[r], cis[r], fv, fi, FCAP, t_u, cnt_v)
            for i in range(KPAD // 16):
                ov[pl.ds(i * 16, 16)] = _splat_f32(NEG_INF)
                oi[pl.ds(i * 16, 16)] = _splat_i32(0)

            def ebody(j, _):
                mvec = fv[pl.ds(0, 16)]
                for i in range(1, FCAP // 16):
                    mvec = jnp.maximum(mvec, fv[pl.ds(i * 16, 16)])
                msp = _splat_f32(1.0) * jnp.max(mvec)
                pos_v = _splat_i32(FCAP)
                for i in range(FCAP // 16):
                    eq = fv[pl.ds(i * 16, 16)] == msp
                    pos_v = jnp.minimum(
                        pos_v, jnp.where(eq, i * 16 + lanes, FCAP))
                pos = _splat_i32(1) * jnp.min(pos_v)
                pos = jnp.minimum(pos, _splat_i32(FCAP - 1))
                idv = plsc.load_gather(fi, [pos])
                jsp = _splat_i32(1) * j
                lane0 = lanes == 0
                plsc.store_scatter(ov, [jsp], msp, mask=lane0)
                plsc.store_scatter(oi, [jsp], idv, mask=lane0)
                plsc.store_scatter(fv, [pos], _splat_f32(NEG_INF),
                                   mask=lane0)
                return 0
            lax.fori_loop(0, K, ebody, 0)

            pltpu.sync_copy(gumbel_hbm.at[pl.ds(row * KPAD, KPAD)], gb)
            zbest = _splat_f32(NEG_INF)
            zs = []
            for i in range(KPAD // 16):
                z = ov[pl.ds(i * 16, 16)] + gb[pl.ds(i * 16, 16)]
                zs.append(z)
                zbest = jnp.maximum(zbest, z)
            msp = _splat_f32(1.0) * jnp.max(zbest)
            pos_v = _splat_i32(KPAD)
            for i in range(KPAD // 16):
                eq = zs[i] == msp
                pos_v = jnp.minimum(pos_v,
                                    jnp.where(eq, i * 16 + lanes, KPAD))
            pos = _splat_i32(1) * jnp.min(pos_v)
            pos = jnp.minimum(pos, _splat_i32(KPAD - 1))
            actb[...] = plsc.load_gather(oi, [pos])

            pltpu.sync_copy(ov, vals_hbm.at[pl.ds(row * KPAD, KPAD)])
            pltpu.sync_copy(actb, act_hbm.at[pl.ds(row * ACTW, ACTW)])

    return k(logits2d, tail_flat, gumbel_flat)


def kernel(logits):
    gkey = jax.random.key(42)
    u = jax.random.uniform(gkey, (B, K), minval=1e-20, maxval=1.0)
    gumbel = -jnp.log(-jnp.log(u))
    gpad = jnp.full((B, KPAD), NEG_INF, dtype=jnp.float32)
    gpad = gpad.at[:, :K].set(gumbel)

    tail_flat = logits[:, NFULL * WC:].reshape(-1)
    vals_flat, act_flat = _sc_topk_sample(logits, tail_flat,
                                          gpad.reshape(-1))
    vals = vals_flat.reshape(B, KPAD)[:, :K]
    act = act_flat.reshape(B, ACTW)[:, 0]
    return act, vals
